# baseline (speedup 1.0000x reference)
import functools

import jax
import jax.numpy as jnp
from jax import lax
from jax.experimental import pallas as pl
from jax.experimental.pallas import tpu as pltpu
from jax.experimental.pallas import tpu_sc as plsc


@functools.cache
def _make_gather(v, d, bsz, hist):
    info = plsc.get_sparse_core_info()
    nc, ns = info.num_cores, info.num_subcores
    nw = nc * ns
    assert bsz % (128 * nw) == 0 and hist % 8 == 0 and d % 8 == 0
    nbc = bsz // 128
    nhr = hist // 8
    nfb = d // 8
    passes = nbc // nw
    mesh = plsc.VectorSubcoreMesh(core_axis_name="c", subcore_axis_name="s")

    @functools.partial(
        pl.kernel,
        mesh=mesh,
        out_type=jax.ShapeDtypeStruct((hist, nfb, nbc, 8, 128), jnp.float32),
        compiler_params=pltpu.CompilerParams(
            use_tc_tiling_on_sc=False, needs_layout_passes=False),
        scratch_types=(
            [pltpu.VMEM((nhr, 8, 128), jnp.int32)]
            + [pltpu.VMEM((256, d), jnp.float32) for _ in range(2)]
            + [pltpu.VMEM((2, nfb, 8, 128), jnp.float32) for _ in range(2)]
            + [pltpu.SemaphoreType.DMA for _ in range(4)]
        ),
    )
    def gather(table_hbm, x4_hbm, out_hbm, idx_v, rb0, rb1, tb0, tb1,
               g0, g1, s0, s1):
        rbufs, tbufs, gsems, ssems = (rb0, rb1), (tb0, tb1), (g0, g1), (s0, s1)
        wid = lax.axis_index("s") * nc + lax.axis_index("c")
        ngrp = hist // 2

        def g_copy(h, p):
            return pltpu.make_async_copy(
                table_hbm.at[idx_v.at[h // 8, h % 8]],
                rbufs[p].at[pl.ds(128 * (h % 2), 128)], gsems[p])

        def g_group(j, p):
            g_copy(2 * j, p).start()
            g_copy(2 * j + 1, p).start()

        def s_copy(j, p, bc):
            return pltpu.make_async_copy(
                tbufs[p], out_hbm.at[pl.ds(2 * j, 2), :, bc], ssems[p])

        def transpose(p):
            rbuf, tbuf = rbufs[p], tbufs[p]
            lanes = lax.iota(jnp.int32, 16)
            for i in range(2):
                for fb in range(nfb):
                    for fi in range(8):
                        col = jnp.full((16,), fb * 8 + fi, jnp.int32)
                        for k in range(8):
                            row0 = 128 * i + 16 * k
                            vec = plsc.load_gather(rbuf, [row0 + lanes, col])
                            tbuf[i, fb, fi, pl.ds(16 * k, 16)] = vec

        def one_pass(bc):
            pltpu.sync_copy(x4_hbm.at[:, bc], idx_v)
            g_group(0, 0)

            def body(g, carry):
                for p in range(2):
                    j = 2 * g + p
                    g_copy(2 * j, p).wait()
                    g_copy(2 * j + 1, p).wait()

                    @pl.when(j + 1 < ngrp)
                    def _():
                        g_group(j + 1, 1 - p)

                    @pl.when(j >= 2)
                    def _():
                        s_copy(j - 2, p, bc).wait()

                    transpose(p)
                    s_copy(j, p, bc).start()
                return carry

            lax.fori_loop(0, ngrp // 2, body, 0, unroll=False)
            s_copy(ngrp - 2, 0, bc).wait()
            s_copy(ngrp - 1, 1, bc).wait()

        for i in range(passes):
            one_pass(wid * passes + i)

    return gather


def kernel(x, emb):
    bsz, hist = x.shape
    v, d = emb.shape
    x4 = x.T.reshape(hist // 8, 8, bsz // 128, 128).transpose(0, 2, 1, 3)
    x4 = x4.astype(jnp.int32)
    out4 = _make_gather(v, d, bsz, hist)(emb, x4)
    return out4.transpose(2, 4, 0, 1, 3).reshape(bsz, hist, d)


# speedup vs baseline: 1.2063x; 1.2063x over previous
import functools

import jax
import jax.numpy as jnp
from jax import lax
from jax.experimental import pallas as pl
from jax.experimental.pallas import tpu as pltpu
from jax.experimental.pallas import tpu_sc as plsc


@functools.cache
def _make_gather(v, d, bsz, hist):
    info = plsc.get_sparse_core_info()
    nc, ns = info.num_cores, info.num_subcores
    nw = nc * ns
    assert bsz % (128 * nw) == 0 and hist % 8 == 0 and d % 8 == 0
    nbc = bsz // 128
    nhr = hist // 8
    nfb = d // 8
    passes = nbc // nw
    mesh = plsc.VectorSubcoreMesh(core_axis_name="c", subcore_axis_name="s")

    @functools.partial(
        pl.kernel,
        mesh=mesh,
        out_type=jax.ShapeDtypeStruct((hist, nfb, nbc, 8, 128), jnp.float32),
        compiler_params=pltpu.CompilerParams(
            use_tc_tiling_on_sc=False, needs_layout_passes=False),
        scratch_types=(
            [pltpu.VMEM((nhr, 8, 128), jnp.int32)]
            + [pltpu.VMEM((256, d), jnp.float32) for _ in range(2)]
            + [pltpu.VMEM((2, nfb, 8, 128), jnp.float32) for _ in range(2)]
            + [pltpu.SemaphoreType.DMA for _ in range(4)]
        ),
    )
    def gather(table_hbm, x4_hbm, out_hbm, idx_v, rb0, rb1, tb0, tb1,
               g0, g1, s0, s1):
        rbufs, tbufs, gsems, ssems = (rb0, rb1), (tb0, tb1), (g0, g1), (s0, s1)
        wid = lax.axis_index("s") * nc + lax.axis_index("c")
        ngrp = hist // 2

        def g_copy(h, p):
            return pltpu.make_async_copy(
                table_hbm.at[idx_v.at[h // 8, h % 8]],
                rbufs[p].at[pl.ds(128 * (h % 2), 128)], gsems[p])

        def g_group(j, p):
            g_copy(2 * j, p).start()
            g_copy(2 * j + 1, p).start()

        def s_copy(j, p, bc):
            return pltpu.make_async_copy(
                tbufs[p], out_hbm.at[pl.ds(2 * j, 2), :, bc], ssems[p])

        def transpose(p):
            rbuf, tbuf = rbufs[p], tbufs[p]
            lanes = lax.iota(jnp.int32, 16)
            rowv = [16 * k + lanes for k in range(8)]
            for i in range(2):
                for fb in range(nfb):
                    for fi in range(8):
                        col = jnp.full((16,), fb * 8 + fi, jnp.int32)
                        vecs = [plsc.load_gather(rbuf, [128 * i + rowv[k], col])
                                for k in range(8)]
                        for k in range(8):
                            tbuf[i, fb, fi, pl.ds(16 * k, 16)] = vecs[k]

        def one_pass(bc):
            pltpu.sync_copy(x4_hbm.at[:, bc], idx_v)
            g_group(0, 0)

            def body(g, carry):
                for p in range(2):
                    j = 2 * g + p
                    g_copy(2 * j, p).wait()
                    g_copy(2 * j + 1, p).wait()

                    @pl.when(j + 1 < ngrp)
                    def _():
                        g_group(j + 1, 1 - p)

                    @pl.when(j >= 2)
                    def _():
                        s_copy(j - 2, p, bc).wait()

                    transpose(p)
                    s_copy(j, p, bc).start()
                return carry

            lax.fori_loop(0, ngrp // 2, body, 0, unroll=False)
            s_copy(ngrp - 2, 0, bc).wait()
            s_copy(ngrp - 1, 1, bc).wait()

        for i in range(passes):
            one_pass(wid * passes + i)

    return gather


def kernel(x, emb):
    bsz, hist = x.shape
    v, d = emb.shape
    x4 = x.T.reshape(hist // 8, 8, bsz // 128, 128).transpose(0, 2, 1, 3)
    x4 = x4.astype(jnp.int32)
    out4 = _make_gather(v, d, bsz, hist)(emb, x4)
    return out4.transpose(2, 4, 0, 1, 3).reshape(bsz, hist, d)
